# baseline (device time: 29622 ns/iter reference)
import jax
import jax.numpy as jnp
from jax import lax
from jax.experimental import pallas as pl
from jax.experimental.pallas import tpu as pltpu


def kernel(Q, K, V):
    b, s, h, d = Q.shape
    bs, hd = b * s, h * d
    qw = hd // 4
    scale = d ** -0.5

    Q2 = Q.reshape(bs, hd)
    K2 = K.reshape(bs, hd)
    V2 = V.reshape(bs, hd)

    def body(q_hbm, k_hbm, v_hbm, o_hbm, qv, kv, vv, rk, rv, ov,
             lsem, send_sems, recv_sems):
        my_x = lax.axis_index("x")
        my_y = lax.axis_index("y")
        my_z = lax.axis_index("z")
        peer_x = (1 - my_x, my_y, my_z)
        nb_y = (my_x, 1 - my_y, my_z)
        nb_z = (my_x, my_y, 1 - my_z)

        j_me = 2 * my_y + my_z
        j_of_slot = (j_me, j_me ^ 2, j_me ^ 1, j_me ^ 3)

        def col(ref, j):
            return ref.at[:, pl.ds(qw * j, qw)]

        def stage(x_hbm, x_vmem, t, sem_i):
            cp = pltpu.make_async_copy(
                col(x_hbm, j_of_slot[t]), x_vmem.at[t], lsem.at[sem_i])
            cp.start()
            return cp

        c_kv0 = stage(k_hbm, kv, 0, 0)
        c_vv0 = stage(v_hbm, vv, 0, 1)
        c_q = [stage(q_hbm, qv, t, 2 + t) for t in range(4)]
        c_k = [c_kv0] + [stage(k_hbm, kv, t, 5 + t) for t in (1, 2, 3)]
        c_v = [c_vv0] + [stage(v_hbm, vv, t, 8 + t) for t in (1, 2, 3)]

        barrier = pltpu.get_barrier_semaphore()
        for nbr in (peer_x, nb_y, nb_z):
            pl.semaphore_signal(barrier, inc=1, device_id=nbr,
                                device_id_type=pl.DeviceIdType.MESH)
        pl.semaphore_wait(barrier, 3)

        def copy(src, dst, sem_i, dev):
            return pltpu.make_async_remote_copy(
                src_ref=src, dst_ref=dst,
                send_sem=send_sems.at[sem_i], recv_sem=recv_sems.at[sem_i],
                device_id=dev, device_id_type=pl.DeviceIdType.MESH,
            )

        C = (0, 1)

        def chk(ref, t, c):
            return ref.at[t, pl.ds(bs // 2 * c, bs // 2)]

        o1k = [copy(chk(kv, 0, c), chk(rk, 0, c), 0 + c, peer_x) for c in C]
        o1v = [copy(chk(vv, 0, c), chk(rv, 0, c), 2 + c, peer_x) for c in C]
        o2ky = [copy(chk(rk, 0, c), chk(rk, 1, c), 4 + c, nb_y) for c in C]
        o2vy = [copy(chk(rv, 0, c), chk(rv, 1, c), 6 + c, nb_y) for c in C]
        o2kz = [copy(chk(rk, 0, c), chk(rk, 2, c), 8 + c, nb_z) for c in C]
        o2vz = [copy(chk(rv, 0, c), chk(rv, 2, c), 10 + c, nb_z) for c in C]
        o3y = [copy(chk(rk, 2, c), chk(rk, 3, c), 12 + c, nb_y) for c in C]
        o3z = [copy(chk(rv, 1, c), chk(rv, 3, c), 14 + c, nb_z) for c in C]

        dummy = kv.at[0, pl.ds(0, bs // 2)]
        i2k = [copy(dummy, chk(rk, 1, c), 4 + c, nb_y) for c in C]
        i2v = [copy(dummy, chk(rv, 1, c), 6 + c, nb_y) for c in C]
        i3k = [copy(dummy, chk(rk, 2, c), 8 + c, nb_z) for c in C]
        i3v = [copy(dummy, chk(rv, 2, c), 10 + c, nb_z) for c in C]
        i4k = [copy(dummy, chk(rk, 3, c), 12 + c, nb_y) for c in C]
        i4v = [copy(dummy, chk(rv, 3, c), 14 + c, nb_z) for c in C]

        def compute_slot(t):
            for bb in range(b):
                r = slice(s * bb, s * bb + s)
                for hh in range(2):
                    csl = slice(d * hh, d * hh + d)
                    q = qv[t, r, csl]
                    s_l = lax.dot_general(
                        q, kv[t, r, csl],
                        (((1,), (1,)), ((), ()))) * scale
                    s_r = lax.dot_general(
                        q, rk[t, r, csl],
                        (((1,), (1,)), ((), ()))) * scale
                    m = jnp.maximum(
                        jnp.max(s_l, axis=1, keepdims=True),
                        jnp.max(s_r, axis=1, keepdims=True),
                    )
                    p_l = jnp.exp(s_l - m)
                    p_r = jnp.exp(s_r - m)
                    denom = (jnp.sum(p_l, axis=1, keepdims=True)
                             + jnp.sum(p_r, axis=1, keepdims=True))
                    acc = (lax.dot_general(p_l, vv[t, r, csl],
                                           (((1,), (0,)), ((), ())))
                           + lax.dot_general(p_r, rv[t, r, csl],
                                             (((1,), (0,)), ((), ()))))
                    ov[t, r, csl] = acc / denom

        def writeback(t):
            cp = pltpu.make_async_copy(
                ov.at[t], col(o_hbm, j_of_slot[t]), lsem.at[12 + t])
            cp.start()
            return cp

        c_kv0.wait()
        c_vv0.wait()
        for c in C:
            o1k[c].start()
            o1v[c].start()
        for c in C:
            o1k[c].wait_recv()
            o2ky[c].start()
            o2kz[c].start()
            o1v[c].wait_recv()
            o2vy[c].start()
            o2vz[c].start()
        c_q[0].wait()
        compute_slot(0)
        w0 = writeback(0)
        i3k[0].wait_recv()
        o3y[0].start()
        i2v[0].wait_recv()
        o3z[0].start()
        i3k[1].wait_recv()
        o3y[1].start()
        i2v[1].wait_recv()
        o3z[1].start()
        i2k[0].wait_recv()
        i2k[1].wait_recv()
        c_q[1].wait()
        c_k[1].wait()
        c_v[1].wait()
        compute_slot(1)
        w1 = writeback(1)
        i3v[0].wait_recv()
        i3v[1].wait_recv()
        c_q[2].wait()
        c_k[2].wait()
        c_v[2].wait()
        compute_slot(2)
        w2 = writeback(2)
        for c in C:
            i4k[c].wait_recv()
            i4v[c].wait_recv()
        c_q[3].wait()
        c_k[3].wait()
        c_v[3].wait()
        compute_slot(3)
        w3 = writeback(3)

        for w in (w0, w1, w2, w3):
            w.wait()
        for dsc in (o1k + o1v + o2ky + o2vy + o2kz + o2vz + o3y + o3z):
            dsc.wait_send()

    out = pl.pallas_call(
        body,
        out_shape=jax.ShapeDtypeStruct((bs, hd), jnp.float32),
        in_specs=[pl.BlockSpec(memory_space=pltpu.MemorySpace.HBM)] * 3,
        out_specs=pl.BlockSpec(memory_space=pltpu.MemorySpace.HBM),
        scratch_shapes=[
            pltpu.VMEM((4, bs, qw), jnp.float32),
            pltpu.VMEM((4, bs, qw), jnp.float32),
            pltpu.VMEM((4, bs, qw), jnp.float32),
            pltpu.VMEM((4, bs, qw), jnp.float32),
            pltpu.VMEM((4, bs, qw), jnp.float32),
            pltpu.VMEM((4, bs, qw), jnp.float32),
            pltpu.SemaphoreType.DMA((16,)),
            pltpu.SemaphoreType.DMA((16,)),
            pltpu.SemaphoreType.DMA((16,)),
        ],
        compiler_params=pltpu.CompilerParams(collective_id=0),
    )(Q2, K2, V2)

    return out.reshape(b, s, h, d)


# device time: 21887 ns/iter; 1.3534x vs baseline; 1.3534x over previous
import jax
import jax.numpy as jnp
from jax import lax
from jax.experimental import pallas as pl
from jax.experimental.pallas import tpu as pltpu


def kernel(Q, K, V):
    b, s, h, d = Q.shape
    bh = b * h
    hq = bh // 4
    hc = hq // 2
    scale = d ** -0.5

    Qt = Q.transpose(0, 2, 1, 3).reshape(bh, s, d)
    Kt = K.transpose(0, 2, 3, 1).reshape(bh, d, s)
    Vt = V.transpose(0, 2, 3, 1).reshape(bh, d, s)

    def body(q_hbm, k_hbm, v_hbm, o_ref, qv, kv, vv, rk, rv,
             lsem, send_sems, recv_sems):
        my_x = lax.axis_index("x")
        my_y = lax.axis_index("y")
        my_z = lax.axis_index("z")
        peer_x = (1 - my_x, my_y, my_z)
        nb_y = (my_x, 1 - my_y, my_z)
        nb_z = (my_x, my_y, 1 - my_z)

        j_me = 2 * my_y + my_z
        j_y = 2 * (1 - my_y) + my_z
        j_z = 2 * my_y + (1 - my_z)
        j_diag = 2 * (1 - my_y) + (1 - my_z)

        def csl(ref, j, c):
            return ref.at[pl.ds(hq * j + hc * c, hc)]

        cq = pltpu.make_async_copy(q_hbm, qv, lsem.at[0])
        ck = pltpu.make_async_copy(k_hbm, kv, lsem.at[1])
        cv = pltpu.make_async_copy(v_hbm, vv, lsem.at[2])
        cq.start()
        ck.start()
        cv.start()

        barrier = pltpu.get_barrier_semaphore()
        for nbr in (peer_x, nb_y, nb_z):
            pl.semaphore_signal(barrier, inc=1, device_id=nbr,
                                device_id_type=pl.DeviceIdType.MESH)
        pl.semaphore_wait(barrier, 3)

        def copy(src, dst, sem_i, dev):
            return pltpu.make_async_remote_copy(
                src_ref=src, dst_ref=dst,
                send_sem=send_sems.at[sem_i], recv_sem=recv_sems.at[sem_i],
                device_id=dev, device_id_type=pl.DeviceIdType.MESH,
            )

        C = (0, 1)
        o1k = [copy(csl(k_hbm, j_me, c), csl(rk, j_me, c), 0 + c, peer_x)
               for c in C]
        o1v = [copy(csl(v_hbm, j_me, c), csl(rv, j_me, c), 2 + c, peer_x)
               for c in C]
        dk = pl.ds(hq * j_diag, 1)
        dv = pl.ds(hq * j_diag, 1)
        o1dk = copy(k_hbm.at[dk], rk.at[dk], 16, peer_x)
        o1dv = copy(v_hbm.at[dv], rv.at[dv], 17, peer_x)
        o2ky = [copy(csl(rk, j_me, c), csl(rk, j_me, c), 4 + c, nb_y)
                for c in C]
        o2vy = [copy(csl(rv, j_me, c), csl(rv, j_me, c), 6 + c, nb_y)
                for c in C]
        o2kz = [copy(csl(rk, j_me, c), csl(rk, j_me, c), 8 + c, nb_z)
                for c in C]
        o2vz = [copy(csl(rv, j_me, c), csl(rv, j_me, c), 10 + c, nb_z)
                for c in C]
        def tail3(ref, j, c):
            return ref.at[pl.ds(hq * j + 1 + c, 1 + c)]

        o3y = [copy(tail3(rk, j_z, c), tail3(rk, j_z, c), 12 + c, nb_y)
               for c in C]
        o3z = [copy(tail3(rv, j_y, c), tail3(rv, j_y, c), 14 + c, nb_z)
               for c in C]

        dummy = kv.at[pl.ds(0, hc)]
        dummy1 = kv.at[pl.ds(0, 1)]
        i2k = [copy(dummy, csl(rk, j_y, c), 4 + c, nb_y) for c in C]
        i2v = [copy(dummy, csl(rv, j_y, c), 6 + c, nb_y) for c in C]
        i3k = [copy(dummy, csl(rk, j_z, c), 8 + c, nb_z) for c in C]
        i3v = [copy(dummy, csl(rv, j_z, c), 10 + c, nb_z) for c in C]
        i4k = [copy(kv.at[pl.ds(0, 1 + c)], tail3(rk, j_diag, c), 12 + c,
                    nb_y) for c in C]
        i4v = [copy(kv.at[pl.ds(0, 1 + c)], tail3(rv, j_diag, c), 14 + c,
                    nb_z) for c in C]
        i1dk = copy(dummy1, rk.at[pl.ds(hq * j_diag, 1)], 16, peer_x)
        i1dv = copy(dummy1, rv.at[pl.ds(hq * j_diag, 1)], 17, peer_x)

        def local_half(i):
            q = qv[i]
            s_l = lax.dot_general(
                q, kv[i], (((1,), (0,)), ((), ()))) * scale
            m_l = jnp.max(s_l, axis=1, keepdims=True)
            p_l = jnp.exp(s_l - m_l)
            l_l = jnp.sum(p_l, axis=1, keepdims=True)
            acc_l = lax.dot_general(p_l, vv[i], (((1,), (1,)), ((), ())))
            return m_l, l_l, acc_l

        def merge_quarter(j, parts):
            for t in range(hq):
                i = hq * j + t
                m_l, l_l, acc_l = parts[t]
                q = qv[i]
                s_r = lax.dot_general(
                    q, rk[i], (((1,), (0,)), ((), ()))) * scale
                m_r = jnp.max(s_r, axis=1, keepdims=True)
                m = jnp.maximum(m_l, m_r)
                p_r = jnp.exp(s_r - m)
                a_l = jnp.exp(m_l - m)
                denom = l_l * a_l + jnp.sum(p_r, axis=1, keepdims=True)
                acc = (acc_l * a_l
                       + lax.dot_general(p_r, rv[i], (((1,), (1,)), ((), ()))))
                o_ref[i] = acc / denom

        o1k[0].start()
        o1v[0].start()
        o1k[1].start()
        o1v[1].start()
        o1dk.start()
        o1dv.start()
        cq.wait()
        ck.wait()
        cv.wait()
        parts = {}
        parts[0] = [local_half(hq * j_me + t) for t in range(hq)]
        parts[1] = [local_half(hq * j_y + t) for t in range(hq)]
        o1k[0].wait_recv()
        o2ky[0].start()
        o2kz[0].start()
        parts[2] = [local_half(hq * j_z + t) for t in range(hq)]
        o1v[0].wait_recv()
        o2vy[0].start()
        o2vz[0].start()
        parts[3] = [local_half(hq * j_diag + t) for t in range(hq)]
        o1k[1].wait_recv()
        o2ky[1].start()
        o2kz[1].start()
        o1v[1].wait_recv()
        o2vy[1].start()
        o2vz[1].start()
        merge_quarter(j_me, parts[0])
        i3k[0].wait_recv()
        o3y[0].start()
        i2v[0].wait_recv()
        o3z[0].start()
        i3k[1].wait_recv()
        o3y[1].start()
        i2v[1].wait_recv()
        o3z[1].start()
        i2k[0].wait_recv()
        i2k[1].wait_recv()
        merge_quarter(j_y, parts[1])
        i3v[0].wait_recv()
        i3v[1].wait_recv()
        merge_quarter(j_z, parts[2])
        i1dk.wait_recv()
        i1dv.wait_recv()
        for c in C:
            i4k[c].wait_recv()
            i4v[c].wait_recv()
        merge_quarter(j_diag, parts[3])

        for dsc in (o1k + o1v + o2ky + o2vy + o2kz + o2vz + o3y + o3z
                    + [o1dk, o1dv]):
            dsc.wait_send()

    out = pl.pallas_call(
        body,
        out_shape=jax.ShapeDtypeStruct((bh, s, d), jnp.float32),
        in_specs=[pl.BlockSpec(memory_space=pltpu.MemorySpace.HBM)] * 3,
        out_specs=pl.BlockSpec(memory_space=pltpu.MemorySpace.VMEM),
        scratch_shapes=[
            pltpu.VMEM((bh, s, d), jnp.float32),
            pltpu.VMEM((bh, d, s), jnp.float32),
            pltpu.VMEM((bh, d, s), jnp.float32),
            pltpu.VMEM((bh, d, s), jnp.float32),
            pltpu.VMEM((bh, d, s), jnp.float32),
            pltpu.SemaphoreType.DMA((3,)),
            pltpu.SemaphoreType.DMA((18,)),
            pltpu.SemaphoreType.DMA((18,)),
        ],
        compiler_params=pltpu.CompilerParams(collective_id=0),
    )(Qt, Kt, Vt)

    return out.reshape(b, h, s, d).transpose(0, 2, 1, 3)
